# baseline (speedup 1.0000x reference)
import functools

import jax
import jax.numpy as jnp
from jax import lax
from jax.experimental import pallas as pl
from jax.experimental.pallas import tpu as pltpu
from jax.experimental.pallas import tpu_sc as plsc

_BLK = 2000


def _sc_hist(dst, n):
    e = dst.shape[0]
    nw = 32
    epw = e // nw
    mesh = plsc.VectorSubcoreMesh(core_axis_name="c", subcore_axis_name="s")

    @functools.partial(
        pl.kernel,
        out_type=jax.ShapeDtypeStruct((nw * n,), jnp.float32),
        mesh=mesh,
        compiler_params=pltpu.CompilerParams(needs_layout_passes=False),
        scratch_types=[
            pltpu.VMEM((n,), jnp.float32),
            pltpu.VMEM((epw,), jnp.int32),
        ],
    )
    def k(dst_hbm, out_hbm, hist_v, idx_v):
        cid = lax.axis_index("c")
        sid = lax.axis_index("s")
        wid = sid * 2 + cid
        zero16 = jnp.zeros((16,), jnp.float32)

        def zbody(i, _):
            hist_v[pl.ds(i * 16, 16)] = zero16
            return 0

        lax.fori_loop(0, n // 16, zbody, 0)
        pltpu.sync_copy(dst_hbm.at[pl.ds(wid * epw, epw)], idx_v)
        ones16 = jnp.ones((16,), jnp.float32)

        def body(i, _):
            idx16 = idx_v[pl.ds(i * 16, 16)]
            plsc.addupdate_scatter(hist_v, [idx16], ones16)
            return 0

        lax.fori_loop(0, epw // 16, body, 0)
        pltpu.sync_copy(hist_v, out_hbm.at[pl.ds(wid * n, n)])

    return k(dst).reshape(nw, n)


def _sc_seg(src2, dst2, table, n):
    rows_tot, ch = src2.shape
    ns = 16
    nch = rows_tot // ns
    ib = 16
    nblk = nch // ib
    rps = 624
    rem = n - ns * rps
    zr = 8
    mesh = plsc.VectorSubcoreMesh(core_axis_name="c", subcore_axis_name="s")

    @functools.partial(
        pl.kernel,
        out_type=jax.ShapeDtypeStruct((2 * n, 128), jnp.float32),
        mesh=mesh,
        scratch_types=[
            pltpu.VMEM((2, ib, ch), jnp.int32),
            pltpu.VMEM((2, ib, ch), jnp.int32),
            pltpu.VMEM((ch, 128), jnp.float32),
            pltpu.VMEM((ch, 128), jnp.float32),
            pltpu.VMEM((zr, 128), jnp.float32),
            pltpu.VMEM_SHARED((n, 128), jnp.float32),
            pltpu.SemaphoreType.DMA,
            pltpu.SemaphoreType.DMA,
            pltpu.SemaphoreType.DMA,
            pltpu.SemaphoreType.DMA,
            pltpu.SemaphoreType.DMA,
        ],
    )
    def k(src_hbm, dst_hbm, table_hbm, out_hbm, src_v, dst_v, rows0, rows1,
          zbuf, acc, sem0, sem1, semi, sems0, sems1):
        cid = lax.axis_index("c")
        sid = lax.axis_index("s")
        zero16 = jnp.zeros((16,), jnp.float32)
        for i in range(zr):
            for j in range(8):
                zbuf[i, pl.ds(j * 16, 16)] = zero16

        def zb(i, _):
            pltpu.sync_copy(zbuf, acc.at[pl.ds(sid * rps + i * zr, zr)])
            return 0

        lax.fori_loop(0, rps // zr, zb, 0)

        @pl.when(sid == ns - 1)
        def _():
            pltpu.sync_copy(zbuf, acc.at[pl.ds(ns * rps, zr)])
            pltpu.sync_copy(zbuf, acc.at[pl.ds(ns * rps + zr, rem - zr)])

        ibase = sid * nch
        pltpu.sync_copy(src_hbm.at[pl.ds(ibase, ib)], src_v.at[0])
        pltpu.sync_copy(dst_hbm.at[pl.ds(ibase, ib)], dst_v.at[0])
        plsc.subcore_barrier()

        tbl = table_hbm.at[pl.ds(pl.multiple_of(cid * n, 8), n)]

        def blk(b, _):
            bb = lax.rem(b, 2)

            @pl.when(b > 0)
            def _():
                pltpu.make_async_copy(
                    src_hbm.at[pl.ds(ibase, ib)], src_v.at[bb], semi).wait()
                pltpu.make_async_copy(
                    dst_hbm.at[pl.ds(ibase, ib)], dst_v.at[bb], semi).wait()

            @pl.when(b < nblk - 1)
            def _():
                nb = lax.rem(b + 1, 2)
                noff = ibase + (b + 1) * ib
                pltpu.async_copy(src_hbm.at[pl.ds(noff, ib)], src_v.at[nb],
                                 semi)
                pltpu.async_copy(dst_hbm.at[pl.ds(noff, ib)], dst_v.at[nb],
                                 semi)

            pltpu.async_copy(tbl.at[src_v.at[bb, 0]], rows0, sem0)
            pltpu.async_copy(tbl.at[src_v.at[bb, 1]], rows1, sem1)

            def body(p, _):
                g0 = p * 2
                pltpu.make_async_copy(tbl.at[src_v.at[bb, g0]], rows0,
                                      sem0).wait()
                pltpu.async_copy(rows0, acc.at[dst_v.at[bb, g0]], sems0,
                                 add=True)
                pltpu.make_async_copy(tbl.at[src_v.at[bb, g0 + 1]], rows1,
                                      sem1).wait()
                pltpu.async_copy(rows1, acc.at[dst_v.at[bb, g0 + 1]], sems1,
                                 add=True)
                pltpu.make_async_copy(rows0, acc.at[dst_v.at[bb, g0]],
                                      sems0).wait()
                pltpu.make_async_copy(rows1, acc.at[dst_v.at[bb, g0 + 1]],
                                      sems1).wait()

                @pl.when(p < ib // 2 - 1)
                def _():
                    pltpu.async_copy(tbl.at[src_v.at[bb, g0 + 2]], rows0,
                                     sem0)
                    pltpu.async_copy(tbl.at[src_v.at[bb, g0 + 3]], rows1,
                                     sem1)

                return 0

            lax.fori_loop(0, ib // 2, body, 0)
            return 0

        lax.fori_loop(0, nblk, blk, 0)
        plsc.subcore_barrier()
        pltpu.sync_copy(acc.at[pl.ds(sid * rps, rps)],
                        out_hbm.at[pl.ds(cid * n + sid * rps, rps)])

        @pl.when(sid == ns - 1)
        def _():
            pltpu.sync_copy(acc.at[pl.ds(ns * rps, rem)],
                            out_hbm.at[pl.ds(cid * n + ns * rps, rem)])

    return k(src2, dst2, table)


def _full(shape):
    return pl.BlockSpec(shape, lambda i: tuple(0 for _ in shape))


def _tc_a(flat, spatial, degp, Wt1, bt1, Wt2, bt2, Wg1, Wa, ba, va):
    n, tin = flat.shape
    nb = n // _BLK

    def body(flat_r, sp_r, degp_r, wt1, bt1r, wt2, bt2r, wg1, wa, bar, var_,
             tf_r, et_r, h1p_r, dis_r):
        deg = 1.0 + jnp.sum(degp_r[0], axis=0)[:, None]
        dis = lax.rsqrt(deg)
        t1 = jnp.maximum(flat_r[...] @ wt1[...] + bt1r[...], 0.0)
        tf = t1 @ wt2[...] + bt2r[...]
        tf_r[...] = tf
        et_r[...] = jnp.tanh(tf @ wa[...] + bar[...]) @ var_[...]
        h1p = dis * (sp_r[...] @ wg1[...])
        h1p_r[0] = h1p[:, :128]
        h1p_r[1] = h1p[:, 128:]
        dis_r[...] = dis

    return pl.pallas_call(
        body,
        grid=(nb,),
        in_specs=[
            pl.BlockSpec((_BLK, tin), lambda i: (i, 0)),
            pl.BlockSpec((_BLK, 128), lambda i: (i, 0)),
            pl.BlockSpec((1, 32, _BLK), lambda i: (i, 0, 0)),
            _full(Wt1.shape), _full(bt1.shape), _full(Wt2.shape),
            _full(bt2.shape), _full(Wg1.shape), _full(Wa.shape),
            _full(ba.shape), _full(va.shape),
        ],
        out_specs=[
            pl.BlockSpec((_BLK, 256), lambda i: (i, 0)),
            pl.BlockSpec((_BLK, 1), lambda i: (i, 0)),
            pl.BlockSpec((2, _BLK, 128), lambda i: (0, i, 0)),
            pl.BlockSpec((_BLK, 1), lambda i: (i, 0)),
        ],
        out_shape=[
            jax.ShapeDtypeStruct((n, 256), jnp.float32),
            jax.ShapeDtypeStruct((n, 1), jnp.float32),
            jax.ShapeDtypeStruct((2, n, 128), jnp.float32),
            jax.ShapeDtypeStruct((n, 1), jnp.float32),
        ],
    )(flat, spatial, degp, Wt1, bt1, Wt2, bt2, Wg1, Wa, ba, va)


def _tc_b(s1, h1p, dis, bg1, Wg2):
    n = dis.shape[0]
    nb = n // _BLK

    def body(s1_r, h1p_r, dis_r, bg1r, wg2, h2p_r):
        dis = dis_r[...]
        sf = jnp.concatenate([s1_r[0], s1_r[1]], axis=1)
        hf = jnp.concatenate([h1p_r[0], h1p_r[1]], axis=1)
        x1 = jnp.maximum(dis * (sf + hf) + bg1r[...], 0.0)
        h2p = dis * (x1 @ wg2[...])
        h2p_r[0] = h2p[:, :128]
        h2p_r[1] = h2p[:, 128:]

    return pl.pallas_call(
        body,
        grid=(nb,),
        in_specs=[
            pl.BlockSpec((2, _BLK, 128), lambda i: (0, i, 0)),
            pl.BlockSpec((2, _BLK, 128), lambda i: (0, i, 0)),
            pl.BlockSpec((_BLK, 1), lambda i: (i, 0)),
            _full(bg1.shape), _full(Wg2.shape),
        ],
        out_specs=pl.BlockSpec((2, _BLK, 128), lambda i: (0, i, 0)),
        out_shape=jax.ShapeDtypeStruct((2, n, 128), jnp.float32),
    )(s1, h1p, dis, bg1, Wg2)


def _tc_c(s2, h2p, dis, tf, et, bg2, Wsp, bsp, Wa, ba, va, Wc1, bc1, Wc2,
          bc2):
    n = dis.shape[0]
    nb = n // _BLK
    c = Wc2.shape[1]

    def body(s2_r, h2p_r, dis_r, tf_r, et_r, bg2r, wsp, bspr, wa, bar, var_,
             wc1, bc1r, wc2, bc2r, out_r):
        dis = dis_r[...]
        sf = jnp.concatenate([s2_r[0], s2_r[1]], axis=1)
        hf = jnp.concatenate([h2p_r[0], h2p_r[1]], axis=1)
        x2 = dis * (sf + hf) + bg2r[...]
        s = jnp.maximum(x2 @ wsp[...] + bspr[...], 0.0)
        t = tf_r[...]
        et = et_r[...]
        es = jnp.tanh(s @ wa[...] + bar[...]) @ var_[...]
        m = jnp.maximum(et, es)
        aet = jnp.exp(et - m)
        aes = jnp.exp(es - m)
        fused = (aet * t + aes * s) / (aet + aes)
        h = jnp.maximum(fused @ wc1[...] + bc1r[...], 0.0)
        out_r[...] = h @ wc2[...] + bc2r[...]

    return pl.pallas_call(
        body,
        grid=(nb,),
        in_specs=[
            pl.BlockSpec((2, _BLK, 128), lambda i: (0, i, 0)),
            pl.BlockSpec((2, _BLK, 128), lambda i: (0, i, 0)),
            pl.BlockSpec((_BLK, 1), lambda i: (i, 0)),
            pl.BlockSpec((_BLK, 256), lambda i: (i, 0)),
            pl.BlockSpec((_BLK, 1), lambda i: (i, 0)),
            _full(bg2.shape), _full(Wsp.shape), _full(bsp.shape),
            _full(Wa.shape), _full(ba.shape), _full(va.shape),
            _full(Wc1.shape), _full(bc1.shape), _full(Wc2.shape),
            _full(bc2.shape),
        ],
        out_specs=pl.BlockSpec((_BLK, c), lambda i: (i, 0)),
        out_shape=jax.ShapeDtypeStruct((n, c), jnp.float32),
    )(s2, h2p, dis, tf, et, bg2, Wsp, bsp, Wa, ba, va, Wc1, bc1, Wc2, bc2)


def kernel(temporal_input, spatial_input, edge_index, Wt1, bt1, Wt2, bt2,
           Wg1, bg1, Wg2, bg2, Wsp, bsp, Wa, ba, va, Wc1, bc1, Wc2, bc2):
    n = spatial_input.shape[0]
    e = edge_index.shape[1]
    ch = 125
    flat = temporal_input.reshape(n, -1)
    src = edge_index[0]
    dst = edge_index[1]
    src2 = src.reshape(e // ch, ch)
    dst2 = dst.reshape(e // ch, ch)
    degp = _sc_hist(dst, n)
    degp = degp.reshape(32, n // _BLK, _BLK).transpose(1, 0, 2)
    tf, et, h1p, dis = _tc_a(flat, spatial_input, degp, Wt1,
                             bt1.reshape(1, -1), Wt2, bt2.reshape(1, -1),
                             Wg1, Wa, ba.reshape(1, -1), va.reshape(-1, 1))
    s1 = _sc_seg(src2, dst2, h1p.reshape(2 * n, 128), n).reshape(2, n, 128)
    h2p = _tc_b(s1, h1p, dis, bg1.reshape(1, -1), Wg2)
    s2 = _sc_seg(src2, dst2, h2p.reshape(2 * n, 128), n).reshape(2, n, 128)
    logits = _tc_c(s2, h2p, dis, tf, et, bg2.reshape(1, -1), Wsp,
                   bsp.reshape(1, -1), Wa, ba.reshape(1, -1),
                   va.reshape(-1, 1), Wc1, bc1.reshape(1, -1), Wc2,
                   bc2.reshape(1, -1))
    return logits


# speedup vs baseline: 1.2944x; 1.2944x over previous
import functools

import jax
import jax.numpy as jnp
from jax import lax
from jax.experimental import pallas as pl
from jax.experimental.pallas import tpu as pltpu
from jax.experimental.pallas import tpu_sc as plsc

_BLK = 2000


def _sc_hist(dst, n):
    e = dst.shape[0]
    nw = 32
    epw = e // nw
    mesh = plsc.VectorSubcoreMesh(core_axis_name="c", subcore_axis_name="s")

    @functools.partial(
        pl.kernel,
        out_type=jax.ShapeDtypeStruct((nw * n,), jnp.float32),
        mesh=mesh,
        compiler_params=pltpu.CompilerParams(needs_layout_passes=False),
        scratch_types=[
            pltpu.VMEM((n,), jnp.float32),
            pltpu.VMEM((epw,), jnp.int32),
        ],
    )
    def k(dst_hbm, out_hbm, hist_v, idx_v):
        cid = lax.axis_index("c")
        sid = lax.axis_index("s")
        wid = sid * 2 + cid
        zero16 = jnp.zeros((16,), jnp.float32)

        def zbody(i, _):
            hist_v[pl.ds(i * 16, 16)] = zero16
            return 0

        lax.fori_loop(0, n // 16, zbody, 0)
        pltpu.sync_copy(dst_hbm.at[pl.ds(wid * epw, epw)], idx_v)
        ones16 = jnp.ones((16,), jnp.float32)

        def body(i, _):
            idx16 = idx_v[pl.ds(i * 16, 16)]
            plsc.addupdate_scatter(hist_v, [idx16], ones16)
            return 0

        lax.fori_loop(0, epw // 16, body, 0)
        pltpu.sync_copy(hist_v, out_hbm.at[pl.ds(wid * n, n)])

    return k(dst).reshape(nw, n)


def _sc_seg(src2, dst2, table, n):
    rows_tot, ch = src2.shape
    ns = 16
    nch = rows_tot // ns
    ib = 32
    nblk = nch // ib
    rps = 624
    rem = n - ns * rps
    zr = 8
    mesh = plsc.VectorSubcoreMesh(core_axis_name="c", subcore_axis_name="s")

    @functools.partial(
        pl.kernel,
        out_type=jax.ShapeDtypeStruct((2 * n, 128), jnp.float32),
        mesh=mesh,
        scratch_types=[
            pltpu.VMEM((2, ib, ch), jnp.int32),
            pltpu.VMEM((2, ib, ch), jnp.int32),
            pltpu.VMEM((ch, 128), jnp.float32),
            pltpu.VMEM((ch, 128), jnp.float32),
            pltpu.VMEM((zr, 128), jnp.float32),
            pltpu.VMEM_SHARED((n, 128), jnp.float32),
            pltpu.SemaphoreType.DMA,
            pltpu.SemaphoreType.DMA,
            pltpu.SemaphoreType.DMA,
            pltpu.SemaphoreType.DMA,
            pltpu.SemaphoreType.DMA,
        ],
    )
    def k(src_hbm, dst_hbm, table_hbm, out_hbm, src_v, dst_v, rows0, rows1,
          zbuf, acc, sem0, sem1, semi, sems0, sems1):
        cid = lax.axis_index("c")
        sid = lax.axis_index("s")
        zero16 = jnp.zeros((16,), jnp.float32)
        for i in range(zr):
            for j in range(8):
                zbuf[i, pl.ds(j * 16, 16)] = zero16

        def zb(i, _):
            pltpu.sync_copy(zbuf, acc.at[pl.ds(sid * rps + i * zr, zr)])
            return 0

        lax.fori_loop(0, rps // zr, zb, 0)

        @pl.when(sid == ns - 1)
        def _():
            pltpu.sync_copy(zbuf, acc.at[pl.ds(ns * rps, zr)])
            pltpu.sync_copy(zbuf, acc.at[pl.ds(ns * rps + zr, rem - zr)])

        ibase = sid * nch
        pltpu.sync_copy(src_hbm.at[pl.ds(ibase, ib)], src_v.at[0])
        pltpu.sync_copy(dst_hbm.at[pl.ds(ibase, ib)], dst_v.at[0])
        plsc.subcore_barrier()

        tbl = table_hbm.at[pl.ds(pl.multiple_of(cid * n, 8), n)]

        def blk(b, _):
            bb = lax.rem(b, 2)

            @pl.when(b > 0)
            def _():
                pltpu.make_async_copy(
                    src_hbm.at[pl.ds(ibase, ib)], src_v.at[bb], semi).wait()
                pltpu.make_async_copy(
                    dst_hbm.at[pl.ds(ibase, ib)], dst_v.at[bb], semi).wait()

            @pl.when(b < nblk - 1)
            def _():
                nb = lax.rem(b + 1, 2)
                noff = ibase + (b + 1) * ib
                pltpu.async_copy(src_hbm.at[pl.ds(noff, ib)], src_v.at[nb],
                                 semi)
                pltpu.async_copy(dst_hbm.at[pl.ds(noff, ib)], dst_v.at[nb],
                                 semi)

            pltpu.async_copy(tbl.at[src_v.at[bb, 0]], rows0, sem0)

            def body(p, _):
                g0 = p * 2
                pltpu.async_copy(tbl.at[src_v.at[bb, g0 + 1]], rows1, sem1)
                pltpu.make_async_copy(tbl.at[src_v.at[bb, g0]], rows0,
                                      sem0).wait()
                pltpu.sync_copy(rows0, acc.at[dst_v.at[bb, g0]], add=True)

                @pl.when(p < ib // 2 - 1)
                def _():
                    pltpu.async_copy(tbl.at[src_v.at[bb, g0 + 2]], rows0,
                                     sem0)

                pltpu.make_async_copy(tbl.at[src_v.at[bb, g0 + 1]], rows1,
                                      sem1).wait()
                pltpu.sync_copy(rows1, acc.at[dst_v.at[bb, g0 + 1]], add=True)
                return 0

            lax.fori_loop(0, ib // 2, body, 0)
            return 0

        lax.fori_loop(0, nblk, blk, 0)
        plsc.subcore_barrier()
        pltpu.sync_copy(acc.at[pl.ds(sid * rps, rps)],
                        out_hbm.at[pl.ds(cid * n + sid * rps, rps)])

        @pl.when(sid == ns - 1)
        def _():
            pltpu.sync_copy(acc.at[pl.ds(ns * rps, rem)],
                            out_hbm.at[pl.ds(cid * n + ns * rps, rem)])

    return k(src2, dst2, table)


def _full(shape):
    return pl.BlockSpec(shape, lambda i: tuple(0 for _ in shape))


def _tc_a(flat, spatial, degp, Wt1, bt1, Wt2, bt2, Wg1, Wa, ba, va):
    n, tin = flat.shape
    nb = n // _BLK

    def body(flat_r, sp_r, degp_r, wt1, bt1r, wt2, bt2r, wg1, wa, bar, var_,
             tf_r, et_r, h1p_r, dis_r):
        deg = 1.0 + jnp.sum(degp_r[0], axis=0)[:, None]
        dis = lax.rsqrt(deg)
        t1 = jnp.maximum(flat_r[...] @ wt1[...] + bt1r[...], 0.0)
        tf = t1 @ wt2[...] + bt2r[...]
        tf_r[...] = tf
        et_r[...] = jnp.tanh(tf @ wa[...] + bar[...]) @ var_[...]
        h1p = dis * (sp_r[...] @ wg1[...])
        h1p_r[0] = h1p[:, :128]
        h1p_r[1] = h1p[:, 128:]
        dis_r[...] = dis

    return pl.pallas_call(
        body,
        grid=(nb,),
        in_specs=[
            pl.BlockSpec((_BLK, tin), lambda i: (i, 0)),
            pl.BlockSpec((_BLK, 128), lambda i: (i, 0)),
            pl.BlockSpec((1, 32, _BLK), lambda i: (i, 0, 0)),
            _full(Wt1.shape), _full(bt1.shape), _full(Wt2.shape),
            _full(bt2.shape), _full(Wg1.shape), _full(Wa.shape),
            _full(ba.shape), _full(va.shape),
        ],
        out_specs=[
            pl.BlockSpec((_BLK, 256), lambda i: (i, 0)),
            pl.BlockSpec((_BLK, 1), lambda i: (i, 0)),
            pl.BlockSpec((2, _BLK, 128), lambda i: (0, i, 0)),
            pl.BlockSpec((_BLK, 1), lambda i: (i, 0)),
        ],
        out_shape=[
            jax.ShapeDtypeStruct((n, 256), jnp.float32),
            jax.ShapeDtypeStruct((n, 1), jnp.float32),
            jax.ShapeDtypeStruct((2, n, 128), jnp.float32),
            jax.ShapeDtypeStruct((n, 1), jnp.float32),
        ],
    )(flat, spatial, degp, Wt1, bt1, Wt2, bt2, Wg1, Wa, ba, va)


def _tc_b(s1, h1p, dis, bg1, Wg2):
    n = dis.shape[0]
    nb = n // _BLK

    def body(s1_r, h1p_r, dis_r, bg1r, wg2, h2p_r):
        dis = dis_r[...]
        sf = jnp.concatenate([s1_r[0], s1_r[1]], axis=1)
        hf = jnp.concatenate([h1p_r[0], h1p_r[1]], axis=1)
        x1 = jnp.maximum(dis * (sf + hf) + bg1r[...], 0.0)
        h2p = dis * (x1 @ wg2[...])
        h2p_r[0] = h2p[:, :128]
        h2p_r[1] = h2p[:, 128:]

    return pl.pallas_call(
        body,
        grid=(nb,),
        in_specs=[
            pl.BlockSpec((2, _BLK, 128), lambda i: (0, i, 0)),
            pl.BlockSpec((2, _BLK, 128), lambda i: (0, i, 0)),
            pl.BlockSpec((_BLK, 1), lambda i: (i, 0)),
            _full(bg1.shape), _full(Wg2.shape),
        ],
        out_specs=pl.BlockSpec((2, _BLK, 128), lambda i: (0, i, 0)),
        out_shape=jax.ShapeDtypeStruct((2, n, 128), jnp.float32),
    )(s1, h1p, dis, bg1, Wg2)


def _tc_c(s2, h2p, dis, tf, et, bg2, Wsp, bsp, Wa, ba, va, Wc1, bc1, Wc2,
          bc2):
    n = dis.shape[0]
    nb = n // _BLK
    c = Wc2.shape[1]

    def body(s2_r, h2p_r, dis_r, tf_r, et_r, bg2r, wsp, bspr, wa, bar, var_,
             wc1, bc1r, wc2, bc2r, out_r):
        dis = dis_r[...]
        sf = jnp.concatenate([s2_r[0], s2_r[1]], axis=1)
        hf = jnp.concatenate([h2p_r[0], h2p_r[1]], axis=1)
        x2 = dis * (sf + hf) + bg2r[...]
        s = jnp.maximum(x2 @ wsp[...] + bspr[...], 0.0)
        t = tf_r[...]
        et = et_r[...]
        es = jnp.tanh(s @ wa[...] + bar[...]) @ var_[...]
        m = jnp.maximum(et, es)
        aet = jnp.exp(et - m)
        aes = jnp.exp(es - m)
        fused = (aet * t + aes * s) / (aet + aes)
        h = jnp.maximum(fused @ wc1[...] + bc1r[...], 0.0)
        out_r[...] = h @ wc2[...] + bc2r[...]

    return pl.pallas_call(
        body,
        grid=(nb,),
        in_specs=[
            pl.BlockSpec((2, _BLK, 128), lambda i: (0, i, 0)),
            pl.BlockSpec((2, _BLK, 128), lambda i: (0, i, 0)),
            pl.BlockSpec((_BLK, 1), lambda i: (i, 0)),
            pl.BlockSpec((_BLK, 256), lambda i: (i, 0)),
            pl.BlockSpec((_BLK, 1), lambda i: (i, 0)),
            _full(bg2.shape), _full(Wsp.shape), _full(bsp.shape),
            _full(Wa.shape), _full(ba.shape), _full(va.shape),
            _full(Wc1.shape), _full(bc1.shape), _full(Wc2.shape),
            _full(bc2.shape),
        ],
        out_specs=pl.BlockSpec((_BLK, c), lambda i: (i, 0)),
        out_shape=jax.ShapeDtypeStruct((n, c), jnp.float32),
    )(s2, h2p, dis, tf, et, bg2, Wsp, bsp, Wa, ba, va, Wc1, bc1, Wc2, bc2)


def kernel(temporal_input, spatial_input, edge_index, Wt1, bt1, Wt2, bt2,
           Wg1, bg1, Wg2, bg2, Wsp, bsp, Wa, ba, va, Wc1, bc1, Wc2, bc2):
    n = spatial_input.shape[0]
    e = edge_index.shape[1]
    ch = 125
    flat = temporal_input.reshape(n, -1)
    src = edge_index[0]
    dst = edge_index[1]
    src2 = src.reshape(e // ch, ch)
    dst2 = dst.reshape(e // ch, ch)
    degp = _sc_hist(dst, n)
    degp = degp.reshape(32, n // _BLK, _BLK).transpose(1, 0, 2)
    tf, et, h1p, dis = _tc_a(flat, spatial_input, degp, Wt1,
                             bt1.reshape(1, -1), Wt2, bt2.reshape(1, -1),
                             Wg1, Wa, ba.reshape(1, -1), va.reshape(-1, 1))
    s1 = _sc_seg(src2, dst2, h1p.reshape(2 * n, 128), n).reshape(2, n, 128)
    h2p = _tc_b(s1, h1p, dis, bg1.reshape(1, -1), Wg2)
    s2 = _sc_seg(src2, dst2, h2p.reshape(2 * n, 128), n).reshape(2, n, 128)
    logits = _tc_c(s2, h2p, dis, tf, et, bg2.reshape(1, -1), Wsp,
                   bsp.reshape(1, -1), Wa, ba.reshape(1, -1),
                   va.reshape(-1, 1), Wc1, bc1.reshape(1, -1), Wc2,
                   bc2.reshape(1, -1))
    return logits
